# baseline (speedup 1.0000x reference)
import functools

import jax
import jax.numpy as jnp
from jax import lax
from jax.experimental import pallas as pl
from jax.experimental.pallas import tpu as pltpu

_N = 10000
_BN_EPS = 1e-5
_BM = 400
_PREP_BM = 1000


def _prep_body(x_ref, gc1_W_ref, bi_W_ref, fc1_W_ref, fc1_b_ref,
               fc2_W_ref, fc2_b_ref, bn_scale_ref, bn_beta_ref, b2_ref,
               s1_ref, t_ref):
    xb = x_ref[...]
    s1_ref[...] = jnp.dot(xb, gc1_W_ref[...], preferred_element_type=jnp.float32)
    bw = bi_W_ref[...]
    e = jnp.dot(xb, bw, preferred_element_type=jnp.float32)
    ss = jnp.dot(xb * xb, bw * bw, preferred_element_type=jnp.float32)
    bi = 0.5 * (e * e - ss)
    h = lax.dot_general(bi, fc1_W_ref[...], (((1,), (1,)), ((), ())),
                        preferred_element_type=jnp.float32)
    h = jnp.maximum(h + fc1_b_ref[...], 0.0)
    h2 = lax.dot_general(h, fc2_W_ref[...], (((1,), (1,)), ((), ())),
                         preferred_element_type=jnp.float32)
    h2 = h2 + fc2_b_ref[...]
    xr = jnp.maximum(h2, 0.0) * bn_scale_ref[...] + bn_beta_ref[...]
    t_ref[...] = jnp.dot(xr, b2_ref[...], preferred_element_type=jnp.float32)


def _pass1_body(adj_ref, s1_ref, t_ref, p1_ref, pt_ref):
    a = adj_ref[...]
    p1_ref[...] = jnp.dot(a, s1_ref[...], preferred_element_type=jnp.float32)
    pt_ref[...] = jnp.dot(a, t_ref[...], preferred_element_type=jnp.float32)


def _pass2_body(adj_ref, p1_ref, pt_ref, gc1_b_ref, gc2_b_ref, a2_ref,
                out_ref, m_ref):
    @pl.when(pl.program_id(0) == 0)
    def _():
        xl = jnp.maximum(p1_ref[...] + gc1_b_ref[...], 0.0)
        m_ref[...] = jnp.dot(xl, a2_ref[...], preferred_element_type=jnp.float32)

    o = jnp.dot(adj_ref[...], m_ref[...], preferred_element_type=jnp.float32)
    o = o + pt_ref[...] + gc2_b_ref[...]
    mx = jnp.max(o, axis=1, keepdims=True)
    lse = jnp.log(jnp.sum(jnp.exp(o - mx), axis=1, keepdims=True))
    out_ref[...] = o - mx - lse


def _full(shape):
    return pl.BlockSpec(shape, lambda i: (0,) * len(shape))


@jax.jit
def kernel(x, adj, gc1_W, gc1_b, gc2_W, gc2_b, bi_W, fc1_W, fc1_b, fc2_W,
           fc2_b, bn_gamma, bn_beta):
    n, nfeat = x.shape
    nhid = gc1_W.shape[1]
    nclass = gc2_W.shape[1]
    nemb2 = bi_W.shape[1]
    nemb = fc1_W.shape[0]

    bn_scale = (bn_gamma / jnp.sqrt(1.0 + _BN_EPS)).reshape(1, -1)
    bn_beta2 = bn_beta.reshape(1, -1)
    fc1_b2 = fc1_b.reshape(1, -1)
    fc2_b2 = fc2_b.reshape(1, -1)
    gc1_b2 = gc1_b.reshape(1, -1)
    gc2_b2 = gc2_b.reshape(1, -1)
    a2 = gc2_W[:nhid, :]
    b2 = gc2_W[nhid:, :]

    grid_prep = n // _PREP_BM
    s1, t = pl.pallas_call(
        _prep_body,
        grid=(grid_prep,),
        in_specs=[
            pl.BlockSpec((_PREP_BM, nfeat), lambda i: (i, 0)),
            _full(gc1_W.shape), _full(bi_W.shape), _full(fc1_W.shape),
            _full(fc1_b2.shape), _full(fc2_W.shape), _full(fc2_b2.shape),
            _full(bn_scale.shape), _full(bn_beta2.shape), _full(b2.shape),
        ],
        out_specs=[
            pl.BlockSpec((_PREP_BM, nhid), lambda i: (i, 0)),
            pl.BlockSpec((_PREP_BM, nclass), lambda i: (i, 0)),
        ],
        out_shape=[
            jax.ShapeDtypeStruct((n, nhid), jnp.float32),
            jax.ShapeDtypeStruct((n, nclass), jnp.float32),
        ],
    )(x, gc1_W, bi_W, fc1_W, fc1_b2, fc2_W, fc2_b2, bn_scale, bn_beta2, b2)

    grid1 = n // _BM
    p1, pt = pl.pallas_call(
        _pass1_body,
        grid=(grid1,),
        in_specs=[
            pl.BlockSpec((_BM, n), lambda i: (i, 0)),
            _full((n, nhid)), _full((n, nclass)),
        ],
        out_specs=[
            pl.BlockSpec((_BM, nhid), lambda i: (i, 0)),
            pl.BlockSpec((_BM, nclass), lambda i: (i, 0)),
        ],
        out_shape=[
            jax.ShapeDtypeStruct((n, nhid), jnp.float32),
            jax.ShapeDtypeStruct((n, nclass), jnp.float32),
        ],
    )(adj, s1, t)

    out = pl.pallas_call(
        _pass2_body,
        grid=(grid1,),
        in_specs=[
            pl.BlockSpec((_BM, n), lambda i: (i, 0)),
            _full((n, nhid)),
            pl.BlockSpec((_BM, nclass), lambda i: (i, 0)),
            _full(gc1_b2.shape), _full(gc2_b2.shape), _full(a2.shape),
        ],
        out_specs=pl.BlockSpec((_BM, nclass), lambda i: (i, 0)),
        out_shape=jax.ShapeDtypeStruct((n, nclass), jnp.float32),
        scratch_shapes=[pltpu.VMEM((n, nclass), jnp.float32)],
    )(adj, p1, pt, gc1_b2, gc2_b2, a2)

    return out


# speedup vs baseline: 1.1187x; 1.1187x over previous
import jax
import jax.numpy as jnp
from jax import lax
from jax.experimental import pallas as pl
from jax.experimental.pallas import tpu as pltpu

_BN_EPS = 1e-5
_BM = 400


def _body(x_ref, adj_ref, gc1_W_ref, bi_W_ref, fc1_W_ref, fc1_b_ref,
          fc2_W_ref, fc2_b_ref, bn_scale_ref, bn_beta_ref,
          a2_ref, b2_ref, gc1_b_ref, gc2_b_ref,
          out_ref, pack_s):
    p = pl.program_id(0)
    i = pl.program_id(1)
    rows = pl.ds(i * _BM, _BM)

    @pl.when((p == 0) & (i == 0))
    def _prep():
        xb = x_ref[...]
        s1 = jnp.dot(xb, gc1_W_ref[...], preferred_element_type=jnp.float32)
        bw = bi_W_ref[...]
        e = jnp.dot(xb, bw, preferred_element_type=jnp.float32)
        ss = jnp.dot(xb * xb, bw * bw, preferred_element_type=jnp.float32)
        bi = 0.5 * (e * e - ss)
        h = lax.dot_general(bi, fc1_W_ref[...], (((1,), (1,)), ((), ())),
                            preferred_element_type=jnp.float32)
        h = jnp.maximum(h + fc1_b_ref[...], 0.0)
        h2 = lax.dot_general(h, fc2_W_ref[...], (((1,), (1,)), ((), ())),
                             preferred_element_type=jnp.float32)
        h2 = h2 + fc2_b_ref[...]
        xr = jnp.maximum(h2, 0.0) * bn_scale_ref[...] + bn_beta_ref[...]
        t = jnp.dot(xr, b2_ref[...], preferred_element_type=jnp.float32)
        pack_s[:, 0:24] = jnp.concatenate([s1, t], axis=1)

    @pl.when(p == 0)
    def _pass1():
        pack_s[rows, 24:48] = jnp.dot(adj_ref[...], pack_s[:, 0:24],
                                      preferred_element_type=jnp.float32)

    @pl.when((p == 1) & (i == 0))
    def _mid():
        xl = jnp.maximum(pack_s[:, 24:32] + gc1_b_ref[...], 0.0)
        pack_s[:, 48:64] = jnp.dot(xl, a2_ref[...],
                                   preferred_element_type=jnp.float32)

    @pl.when(p == 1)
    def _pass2():
        o = jnp.dot(adj_ref[...], pack_s[:, 48:64],
                    preferred_element_type=jnp.float32)
        o = o + pack_s[rows, 32:48] + gc2_b_ref[...]
        mx = jnp.max(o, axis=1, keepdims=True)
        lse = jnp.log(jnp.sum(jnp.exp(o - mx), axis=1, keepdims=True))
        out_ref[...] = o - mx - lse


def _full(shape):
    return pl.BlockSpec(shape, lambda p, i: (0,) * len(shape))


@jax.jit
def kernel(x, adj, gc1_W, gc1_b, gc2_W, gc2_b, bi_W, fc1_W, fc1_b, fc2_W,
           fc2_b, bn_gamma, bn_beta):
    n, nfeat = x.shape
    nhid = gc1_W.shape[1]
    nclass = gc2_W.shape[1]

    bn_scale = (bn_gamma / jnp.sqrt(1.0 + _BN_EPS)).reshape(1, -1)
    bn_beta2 = bn_beta.reshape(1, -1)
    fc1_b2 = fc1_b.reshape(1, -1)
    fc2_b2 = fc2_b.reshape(1, -1)
    gc1_b2 = gc1_b.reshape(1, -1)
    gc2_b2 = gc2_b.reshape(1, -1)
    a2 = gc2_W[:nhid, :]
    b2 = gc2_W[nhid:, :]

    out = pl.pallas_call(
        _body,
        grid=(2, n // _BM),
        in_specs=[
            _full(x.shape),
            pl.BlockSpec((_BM, n), lambda p, i: (i, 0)),
            _full(gc1_W.shape), _full(bi_W.shape), _full(fc1_W.shape),
            _full(fc1_b2.shape), _full(fc2_W.shape), _full(fc2_b2.shape),
            _full(bn_scale.shape), _full(bn_beta2.shape),
            _full(a2.shape), _full(b2.shape),
            _full(gc1_b2.shape), _full(gc2_b2.shape),
        ],
        out_specs=pl.BlockSpec((_BM, nclass), lambda p, i: (i, 0)),
        out_shape=jax.ShapeDtypeStruct((n, nclass), jnp.float32),
        scratch_shapes=[
            pltpu.VMEM((n, 64), jnp.float32),
        ],
    )(x, adj, gc1_W, bi_W, fc1_W, fc1_b2, fc2_W, fc2_b2, bn_scale, bn_beta2,
      a2, b2, gc1_b2, gc2_b2)

    return out
